# Initial kernel scaffold; baseline (speedup 1.0000x reference)
import functools

import jax
import jax.numpy as jnp
from jax import lax
from jax.experimental import pallas as pl
from jax.experimental.pallas import tpu as pltpu
from jax.experimental.pallas import tpu_sc as plsc


def _make_pool_kernel(B, V, D, n_ids):
    info = plsc.get_sparse_core_info()
    NC, NS, L = info.num_cores, info.num_subcores, info.num_lanes
    NW = NC * NS
    assert D == 2 * L and n_ids == 10
    assert B % NW == 0
    b_per_w = B // NW
    CHUNK = 128
    assert b_per_w % CHUNK == 0
    n_chunks = b_per_w // CHUNK
    ids_per_chunk = CHUNK * n_ids
    assert ids_per_chunk % 128 == 0
    idx_rows = ids_per_chunk // 128

    mesh = plsc.VectorSubcoreMesh(core_axis_name="c", subcore_axis_name="s")

    @functools.partial(
        pl.kernel,
        mesh=mesh,
        out_type=jax.ShapeDtypeStruct((B, 2 * D), jnp.float32),
        scratch_types=[
            pltpu.VMEM((idx_rows, 128), jnp.int32),
            pltpu.VMEM((ids_per_chunk, D), jnp.float32),
            pltpu.VMEM((CHUNK, 2 * D), jnp.float32),
            pltpu.SemaphoreType.DMA,
        ],
    )
    def pool_kernel(ids_hbm, table_hbm, out_hbm, idx_v, rows_v, pool_v, sem):
        wid = lax.axis_index("s") * NC + lax.axis_index("c")

        for c in range(n_chunks):
            idx_row_base = wid * (n_chunks * idx_rows) + c * idx_rows
            pltpu.sync_copy(ids_hbm.at[pl.ds(idx_row_base, idx_rows)], idx_v)
            handles = []
            for j in range(idx_rows):
                handles.append(
                    pltpu.async_copy(
                        table_hbm.at[idx_v.at[j]],
                        rows_v.at[pl.ds(j * 128, 128)],
                        sem,
                    )
                )
            for h in handles:
                h.wait()

            def body(i, carry):
                i10 = i * n_ids
                for t in range(2):
                    for hh in range(2):
                        acc = rows_v[i10 + 5 * t, pl.ds(hh * L, L)]
                        for j in range(1, 5):
                            acc = acc + rows_v[i10 + 5 * t + j, pl.ds(hh * L, L)]
                        pool_v[i, pl.ds(t * D + hh * L, L)] = acc * 0.2
                return carry

            lax.fori_loop(0, CHUNK, body, 0)

            out_base = wid * b_per_w + c * CHUNK
            pltpu.sync_copy(pool_v, out_hbm.at[pl.ds(out_base, CHUNK)])

    return pool_kernel


def _mlp_body(p_ref, e_ref, w1a_ref, w1b_ref, b1_ref, w2_ref, b2_ref, o_ref):
    h = jnp.dot(p_ref[...], w1a_ref[...], preferred_element_type=jnp.float32)
    h = h + jnp.dot(e_ref[...], w1b_ref[...], preferred_element_type=jnp.float32)
    h = jnp.maximum(h + b1_ref[...], 0.0)
    o_ref[...] = jnp.sum(h * w2_ref[...], axis=1, keepdims=True) + b2_ref[0]


def _mlp(pooled, extras64, W1a, W1b, b1r, w2t, b2, Bt=1024):
    B, F = pooled.shape
    H = W1a.shape[1]
    grid = (B // Bt,)
    return pl.pallas_call(
        _mlp_body,
        grid=grid,
        in_specs=[
            pl.BlockSpec((Bt, F), lambda i: (i, 0)),
            pl.BlockSpec((Bt, F), lambda i: (i, 0)),
            pl.BlockSpec((F, H), lambda i: (0, 0)),
            pl.BlockSpec((F, H), lambda i: (0, 0)),
            pl.BlockSpec((1, H), lambda i: (0, 0)),
            pl.BlockSpec((1, H), lambda i: (0, 0)),
            pl.BlockSpec(memory_space=pltpu.SMEM),
        ],
        out_specs=pl.BlockSpec((Bt, 1), lambda i: (i, 0)),
        out_shape=jax.ShapeDtypeStruct((B, 1), jnp.float32),
    )(pooled, extras64, W1a, W1b, b1r, w2t, b2)


def kernel(radiant_ids, dire_ids, avg_rank_tiers, num_rank_tiers, durations,
           emb_table, W1, b1, W2, b2):
    B = radiant_ids.shape[0]
    V, D = emb_table.shape
    H = W1.shape[1]

    ids = jnp.concatenate(
        [radiant_ids.astype(jnp.int32), dire_ids.astype(jnp.int32)], axis=1)
    ids2d = ids.reshape(B * 10 // 128, 128)

    pooled = _make_pool_kernel(B, V, D, 10)(ids2d, emb_table)

    extras = jnp.stack([avg_rank_tiers, num_rank_tiers, durations], axis=1)
    extras64 = jnp.pad(extras, ((0, 0), (0, 2 * D - 3)))
    W1a = W1[: 2 * D]
    W1b = jnp.pad(W1[2 * D:], ((0, 2 * D - 3), (0, 0)))

    logit = _mlp(pooled, extras64, W1a, W1b,
                 b1.reshape(1, H), W2.reshape(1, H), b2)
    return logit.reshape(B)


# speedup vs baseline: 1.0348x; 1.0348x over previous
import functools

import jax
import jax.numpy as jnp
from jax import lax
from jax.experimental import pallas as pl
from jax.experimental.pallas import tpu as pltpu
from jax.experimental.pallas import tpu_sc as plsc


def _make_pool_kernel(B, V, D, n_ids):
    info = plsc.get_sparse_core_info()
    NC, NS, L = info.num_cores, info.num_subcores, info.num_lanes
    NW = NC * NS
    assert D == 2 * L and n_ids == 10
    assert B % NW == 0
    b_per_w = B // NW
    CHUNK = 128
    assert b_per_w % CHUNK == 0
    n_chunks = b_per_w // CHUNK
    ids_per_chunk = CHUNK * n_ids
    assert ids_per_chunk % 128 == 0
    idx_rows = ids_per_chunk // 128
    idx_rows_w = n_chunks * idx_rows
    assert idx_rows_w % 8 == 0

    mesh = plsc.VectorSubcoreMesh(core_axis_name="c", subcore_axis_name="s")

    @functools.partial(
        pl.kernel,
        mesh=mesh,
        compiler_params=pltpu.CompilerParams(use_tc_tiling_on_sc=False),
        out_type=jax.ShapeDtypeStruct((B, 2 * D), jnp.float32),
        scratch_types=[
            pltpu.VMEM((idx_rows_w, 128), jnp.int32),
            pltpu.VMEM((ids_per_chunk, D), jnp.float32),
            pltpu.VMEM((CHUNK, 2 * D), jnp.float32),
            pltpu.SemaphoreType.DMA,
        ],
    )
    def pool_kernel(ids_hbm, table_hbm, out_hbm, idx_v, rows_v, pool_v, sem):
        wid = lax.axis_index("s") * NC + lax.axis_index("c")
        pltpu.sync_copy(ids_hbm.at[pl.ds(wid * idx_rows_w, idx_rows_w)], idx_v)

        for c in range(n_chunks):
            handles = []
            for j in range(idx_rows):
                handles.append(
                    pltpu.async_copy(
                        table_hbm.at[idx_v.at[c * idx_rows + j]],
                        rows_v.at[pl.ds(j * 128, 128)],
                        sem,
                    )
                )
            for h in handles:
                h.wait()

            def body(i, carry):
                i10 = i * n_ids
                for t in range(2):
                    for hh in range(2):
                        acc = rows_v[i10 + 5 * t, pl.ds(hh * L, L)]
                        for j in range(1, 5):
                            acc = acc + rows_v[i10 + 5 * t + j, pl.ds(hh * L, L)]
                        pool_v[i, pl.ds(t * D + hh * L, L)] = acc * 0.2
                return carry

            lax.fori_loop(0, CHUNK, body, 0)

            out_base = wid * b_per_w + c * CHUNK
            pltpu.sync_copy(pool_v, out_hbm.at[pl.ds(out_base, CHUNK)])

    return pool_kernel


def _mlp_body(p_ref, e_ref, w1a_ref, w1b_ref, b1_ref, w2_ref, b2_ref, o_ref):
    h = jnp.dot(p_ref[...], w1a_ref[...], preferred_element_type=jnp.float32)
    h = h + jnp.dot(e_ref[...], w1b_ref[...], preferred_element_type=jnp.float32)
    h = jnp.maximum(h + b1_ref[...], 0.0)
    o_ref[...] = jnp.sum(h * w2_ref[...], axis=1, keepdims=True) + b2_ref[0]


def _mlp(pooled, extras64, W1a, W1b, b1r, w2t, b2, Bt=1024):
    B, F = pooled.shape
    H = W1a.shape[1]
    grid = (B // Bt,)
    return pl.pallas_call(
        _mlp_body,
        grid=grid,
        in_specs=[
            pl.BlockSpec((Bt, F), lambda i: (i, 0)),
            pl.BlockSpec((Bt, F), lambda i: (i, 0)),
            pl.BlockSpec((F, H), lambda i: (0, 0)),
            pl.BlockSpec((F, H), lambda i: (0, 0)),
            pl.BlockSpec((1, H), lambda i: (0, 0)),
            pl.BlockSpec((1, H), lambda i: (0, 0)),
            pl.BlockSpec(memory_space=pltpu.SMEM),
        ],
        out_specs=pl.BlockSpec((Bt, 1), lambda i: (i, 0)),
        out_shape=jax.ShapeDtypeStruct((B, 1), jnp.float32),
    )(pooled, extras64, W1a, W1b, b1r, w2t, b2)


def kernel(radiant_ids, dire_ids, avg_rank_tiers, num_rank_tiers, durations,
           emb_table, W1, b1, W2, b2):
    B = radiant_ids.shape[0]
    V, D = emb_table.shape
    H = W1.shape[1]

    ids = jnp.concatenate(
        [radiant_ids.astype(jnp.int32), dire_ids.astype(jnp.int32)], axis=1)
    ids2d = ids.reshape(B * 10 // 128, 128)

    pooled = _make_pool_kernel(B, V, D, 10)(ids2d, emb_table)

    extras = jnp.stack([avg_rank_tiers, num_rank_tiers, durations], axis=1)
    extras64 = jnp.pad(extras, ((0, 0), (0, 2 * D - 3)))
    W1a = W1[: 2 * D]
    W1b = jnp.pad(W1[2 * D:], ((0, 2 * D - 3), (0, 0)))

    logit = _mlp(pooled, extras64, W1a, W1b,
                 b1.reshape(1, H), W2.reshape(1, H), b2)
    return logit.reshape(B)
